# Initial kernel scaffold; baseline (speedup 1.0000x reference)
import functools

import jax
import jax.numpy as jnp
from jax import lax
from jax.experimental import pallas as pl
from jax.experimental.pallas import tpu as pltpu
from jax.experimental.pallas import tpu_sc as plsc

N = 10000
NPAD = 10240
E = 320000
K = 128
CH = 80
NW = 32
EPAD = NW * CH * K
RPT = NPAD // 16
DH = 64
BLK = 1024
NBLK = NPAD // BLK

_mesh = plsc.VectorSubcoreMesh(core_axis_name="c", subcore_axis_name="s")


@functools.partial(
    pl.kernel,
    out_type=jax.ShapeDtypeStruct((2, NPAD, 16), jnp.float32),
    mesh=_mesh,
    scratch_types=[
        pltpu.VMEM_SHARED((NPAD, 16), jnp.float32),
        pltpu.VMEM((CH, K), jnp.int32),
        pltpu.VMEM((K, 16), jnp.float32),
    ],
)
def _sc_degree(dst_hbm, zeros_hbm, ones_hbm, out_hbm, acc_sh, dst_v, ones_v):
    c = lax.axis_index("c")
    s = lax.axis_index("s")
    wid = c * 16 + s
    pltpu.sync_copy(dst_hbm.at[wid], dst_v)
    pltpu.sync_copy(ones_hbm, ones_v)
    pltpu.sync_copy(zeros_hbm.at[pl.ds(s * RPT, RPT)],
                    acc_sh.at[pl.ds(s * RPT, RPT)])
    plsc.subcore_barrier()

    def body(j, carry):
        pltpu.sync_copy(ones_v, acc_sh.at[dst_v.at[j]], add=True)
        return carry

    lax.fori_loop(0, CH, body, 0)
    plsc.subcore_barrier()
    pltpu.sync_copy(acc_sh.at[pl.ds(s * RPT, RPT)],
                    out_hbm.at[c, pl.ds(s * RPT, RPT)])


@functools.partial(
    pl.kernel,
    out_type=jax.ShapeDtypeStruct((2, NPAD, DH), jnp.float32),
    mesh=_mesh,
    scratch_types=[
        pltpu.VMEM_SHARED((NPAD, DH), jnp.float32),
        pltpu.VMEM((CH, K), jnp.int32),
        pltpu.VMEM((CH, K), jnp.int32),
        pltpu.VMEM((K, DH), jnp.float32),
        pltpu.SemaphoreType.DMA,
    ],
)
def _sc_scatter(g_hbm, src_hbm, dst_hbm, zeros_hbm, out_hbm,
                acc_sh, src_v, dst_v, rows_v, sem):
    c = lax.axis_index("c")
    s = lax.axis_index("s")
    wid = c * 16 + s
    pltpu.sync_copy(src_hbm.at[wid], src_v)
    pltpu.sync_copy(dst_hbm.at[wid], dst_v)
    pltpu.sync_copy(zeros_hbm.at[pl.ds(s * RPT, RPT)],
                    acc_sh.at[pl.ds(s * RPT, RPT)])
    plsc.subcore_barrier()

    def body(j, carry):
        pltpu.async_copy(g_hbm.at[src_v.at[j]], rows_v, sem).wait()
        pltpu.sync_copy(rows_v, acc_sh.at[dst_v.at[j]], add=True)
        return carry

    lax.fori_loop(0, CH, body, 0)
    plsc.subcore_barrier()
    pltpu.sync_copy(acc_sh.at[pl.ds(s * RPT, RPT)],
                    out_hbm.at[c, pl.ds(s * RPT, RPT)])


def _tc_pre_body(deg0_ref, deg1_ref, x_ref, w_ref, dinv_ref, g_ref):
    i = pl.program_id(0)
    deg = deg0_ref[:, 0:1] + deg1_ref[:, 0:1] + 1.0
    rows = lax.broadcasted_iota(jnp.int32, (BLK, 1), 0) + i * BLK
    dinv = jnp.where(rows < N, lax.rsqrt(deg), 0.0)
    dinv_ref[...] = dinv
    g_ref[...] = jnp.dot(x_ref[...], w_ref[...],
                         preferred_element_type=jnp.float32) * dinv


def _tc_pre(deg0, deg1, x, w1):
    din = x.shape[1]
    return pl.pallas_call(
        _tc_pre_body,
        grid=(NBLK,),
        in_specs=[
            pl.BlockSpec((BLK, 16), lambda i: (i, 0)),
            pl.BlockSpec((BLK, 16), lambda i: (i, 0)),
            pl.BlockSpec((BLK, din), lambda i: (i, 0)),
            pl.BlockSpec((din, DH), lambda i: (0, 0)),
        ],
        out_specs=[
            pl.BlockSpec((BLK, 1), lambda i: (i, 0)),
            pl.BlockSpec((BLK, DH), lambda i: (i, 0)),
        ],
        out_shape=[
            jax.ShapeDtypeStruct((NPAD, 1), jnp.float32),
            jax.ShapeDtypeStruct((NPAD, DH), jnp.float32),
        ],
    )(deg0, deg1, x, w1)


def _tc_mid_body(s0_ref, s1_ref, g_ref, dinv_ref, b_ref, w_ref, gout_ref):
    dinv = dinv_ref[...]
    a = jnp.maximum(
        dinv * (s0_ref[...] + s1_ref[...] + g_ref[...]) + b_ref[...], 0.0)
    gout_ref[...] = jnp.dot(a, w_ref[...],
                            preferred_element_type=jnp.float32) * dinv


def _tc_mid(s0, s1, g, dinv, b, w2):
    return pl.pallas_call(
        _tc_mid_body,
        grid=(NBLK,),
        in_specs=[
            pl.BlockSpec((BLK, DH), lambda i: (i, 0)),
            pl.BlockSpec((BLK, DH), lambda i: (i, 0)),
            pl.BlockSpec((BLK, DH), lambda i: (i, 0)),
            pl.BlockSpec((BLK, 1), lambda i: (i, 0)),
            pl.BlockSpec((1, DH), lambda i: (0, 0)),
            pl.BlockSpec((DH, DH), lambda i: (0, 0)),
        ],
        out_specs=pl.BlockSpec((BLK, DH), lambda i: (i, 0)),
        out_shape=jax.ShapeDtypeStruct((NPAD, DH), jnp.float32),
    )(s0, s1, g, dinv, b, w2)


def _tc_fin_body(s0_ref, s1_ref, g_ref, dinv_ref, b_ref, w_ref, bf_ref,
                 out_ref):
    a = jnp.maximum(
        dinv_ref[...] * (s0_ref[...] + s1_ref[...] + g_ref[...]) + b_ref[...],
        0.0)
    out_ref[...] = jnp.dot(a, w_ref[...],
                           preferred_element_type=jnp.float32) + bf_ref[...]


def _tc_fin(s0, s1, g, dinv, b, wfc, bfc):
    dout = wfc.shape[1]
    return pl.pallas_call(
        _tc_fin_body,
        grid=(NBLK,),
        in_specs=[
            pl.BlockSpec((BLK, DH), lambda i: (i, 0)),
            pl.BlockSpec((BLK, DH), lambda i: (i, 0)),
            pl.BlockSpec((BLK, DH), lambda i: (i, 0)),
            pl.BlockSpec((BLK, 1), lambda i: (i, 0)),
            pl.BlockSpec((1, DH), lambda i: (0, 0)),
            pl.BlockSpec((DH, dout), lambda i: (0, 0)),
            pl.BlockSpec((1, dout), lambda i: (0, 0)),
        ],
        out_specs=pl.BlockSpec((BLK, dout), lambda i: (i, 0)),
        out_shape=jax.ShapeDtypeStruct((NPAD, dout), jnp.float32),
    )(s0, s1, g, dinv, b, wfc, bfc)


def kernel(x, edge_index, W1, b1, W2, b2, Wfc, bfc):
    src = edge_index[0].astype(jnp.int32)
    dst = edge_index[1].astype(jnp.int32)
    pad = jnp.full((EPAD - E,), N, dtype=jnp.int32)
    srcp = jnp.concatenate([src, pad]).reshape(NW, CH, K)
    dstp = jnp.concatenate([dst, pad]).reshape(NW, CH, K)
    xpad = jnp.pad(x, ((0, NPAD - N), (0, 0)))
    zeros16 = jnp.zeros((NPAD, 16), jnp.float32)
    zeros64 = jnp.zeros((NPAD, DH), jnp.float32)
    ones16 = jnp.ones((K, 16), jnp.float32)
    b1r = b1.reshape(1, DH)
    b2r = b2.reshape(1, DH)
    bfcr = bfc.reshape(1, -1)

    degp = _sc_degree(dstp, zeros16, ones16)
    dinv, g1 = _tc_pre(degp[0], degp[1], xpad, W1)
    s1 = _sc_scatter(g1, srcp, dstp, zeros64)
    g2 = _tc_mid(s1[0], s1[1], g1, dinv, b1r, W2)
    s2 = _sc_scatter(g2, srcp, dstp, zeros64)
    out = _tc_fin(s2[0], s2[1], g2, dinv, b2r, Wfc, bfcr)
    return out[:N]


# speedup vs baseline: 13.9632x; 13.9632x over previous
import functools

import jax
import jax.numpy as jnp
from jax import lax
from jax.experimental import pallas as pl
from jax.experimental.pallas import tpu as pltpu
from jax.experimental.pallas import tpu_sc as plsc

N = 10000
NPAD = 10240
E = 320000
K = 128
CH = 80
NW = 32
EPAD = NW * CH * K
RPT = NPAD // 16
DH = 64
BLK = 1024
NBLK = NPAD // BLK

_mesh = plsc.VectorSubcoreMesh(core_axis_name="c", subcore_axis_name="s")
_sc_params = pltpu.CompilerParams(use_tc_tiling_on_sc=False)


@functools.partial(
    pl.kernel,
    out_type=jax.ShapeDtypeStruct((2, NPAD, 16), jnp.float32),
    mesh=_mesh,
    scratch_types=[
        pltpu.VMEM_SHARED((NPAD, 16), jnp.float32),
        pltpu.VMEM((CH, K), jnp.int32),
        pltpu.VMEM((K, 16), jnp.float32),
    ],
    compiler_params=_sc_params,
)
def _sc_degree(dst_hbm, zeros_hbm, ones_hbm, out_hbm, acc_sh, dst_v, ones_v):
    c = lax.axis_index("c")
    s = lax.axis_index("s")
    wid = c * 16 + s
    pltpu.sync_copy(dst_hbm.at[wid], dst_v)
    pltpu.sync_copy(ones_hbm, ones_v)
    pltpu.sync_copy(zeros_hbm.at[pl.ds(s * RPT, RPT)],
                    acc_sh.at[pl.ds(s * RPT, RPT)])
    plsc.subcore_barrier()

    def body(j, carry):
        pltpu.sync_copy(ones_v, acc_sh.at[dst_v.at[j]], add=True)
        return carry

    lax.fori_loop(0, CH, body, 0)
    plsc.subcore_barrier()
    pltpu.sync_copy(acc_sh.at[pl.ds(s * RPT, RPT)],
                    out_hbm.at[c, pl.ds(s * RPT, RPT)])


@functools.partial(
    pl.kernel,
    out_type=jax.ShapeDtypeStruct((2, NPAD, DH), jnp.float32),
    mesh=_mesh,
    scratch_types=[
        pltpu.VMEM_SHARED((NPAD, DH), jnp.float32),
        pltpu.VMEM((CH, K), jnp.int32),
        pltpu.VMEM((CH, K), jnp.int32),
        pltpu.VMEM((K, DH), jnp.float32),
        pltpu.SemaphoreType.DMA,
    ],
    compiler_params=_sc_params,
)
def _sc_scatter(g_hbm, src_hbm, dst_hbm, zeros_hbm, out_hbm,
                acc_sh, src_v, dst_v, rows_v, sem):
    c = lax.axis_index("c")
    s = lax.axis_index("s")
    wid = c * 16 + s
    pltpu.sync_copy(src_hbm.at[wid], src_v)
    pltpu.sync_copy(dst_hbm.at[wid], dst_v)
    pltpu.sync_copy(zeros_hbm.at[pl.ds(s * RPT, RPT)],
                    acc_sh.at[pl.ds(s * RPT, RPT)])
    plsc.subcore_barrier()

    def body(j, carry):
        pltpu.async_copy(g_hbm.at[src_v.at[j]], rows_v, sem).wait()
        pltpu.sync_copy(rows_v, acc_sh.at[dst_v.at[j]], add=True)
        return carry

    lax.fori_loop(0, CH, body, 0)
    plsc.subcore_barrier()
    pltpu.sync_copy(acc_sh.at[pl.ds(s * RPT, RPT)],
                    out_hbm.at[c, pl.ds(s * RPT, RPT)])


def _tc_pre_body(deg0_ref, deg1_ref, x_ref, w_ref, dinv_ref, g_ref):
    i = pl.program_id(0)
    deg = deg0_ref[:, 0:1] + deg1_ref[:, 0:1] + 1.0
    rows = lax.broadcasted_iota(jnp.int32, (BLK, 1), 0) + i * BLK
    dinv = jnp.where(rows < N, lax.rsqrt(deg), 0.0)
    dinv_ref[...] = dinv
    g_ref[...] = jnp.dot(x_ref[...], w_ref[...],
                         preferred_element_type=jnp.float32) * dinv


def _tc_pre(deg0, deg1, x, w1):
    din = x.shape[1]
    return pl.pallas_call(
        _tc_pre_body,
        grid=(NBLK,),
        in_specs=[
            pl.BlockSpec((BLK, 16), lambda i: (i, 0)),
            pl.BlockSpec((BLK, 16), lambda i: (i, 0)),
            pl.BlockSpec((BLK, din), lambda i: (i, 0)),
            pl.BlockSpec((din, DH), lambda i: (0, 0)),
        ],
        out_specs=[
            pl.BlockSpec((BLK, 1), lambda i: (i, 0)),
            pl.BlockSpec((BLK, DH), lambda i: (i, 0)),
        ],
        out_shape=[
            jax.ShapeDtypeStruct((NPAD, 1), jnp.float32),
            jax.ShapeDtypeStruct((NPAD, DH), jnp.float32),
        ],
    )(deg0, deg1, x, w1)


def _tc_mid_body(s0_ref, s1_ref, g_ref, dinv_ref, b_ref, w_ref, gout_ref):
    dinv = dinv_ref[...]
    a = jnp.maximum(
        dinv * (s0_ref[...] + s1_ref[...] + g_ref[...]) + b_ref[...], 0.0)
    gout_ref[...] = jnp.dot(a, w_ref[...],
                            preferred_element_type=jnp.float32) * dinv


def _tc_mid(s0, s1, g, dinv, b, w2):
    return pl.pallas_call(
        _tc_mid_body,
        grid=(NBLK,),
        in_specs=[
            pl.BlockSpec((BLK, DH), lambda i: (i, 0)),
            pl.BlockSpec((BLK, DH), lambda i: (i, 0)),
            pl.BlockSpec((BLK, DH), lambda i: (i, 0)),
            pl.BlockSpec((BLK, 1), lambda i: (i, 0)),
            pl.BlockSpec((1, DH), lambda i: (0, 0)),
            pl.BlockSpec((DH, DH), lambda i: (0, 0)),
        ],
        out_specs=pl.BlockSpec((BLK, DH), lambda i: (i, 0)),
        out_shape=jax.ShapeDtypeStruct((NPAD, DH), jnp.float32),
    )(s0, s1, g, dinv, b, w2)


def _tc_fin_body(s0_ref, s1_ref, g_ref, dinv_ref, b_ref, w_ref, bf_ref,
                 out_ref):
    a = jnp.maximum(
        dinv_ref[...] * (s0_ref[...] + s1_ref[...] + g_ref[...]) + b_ref[...],
        0.0)
    out_ref[...] = jnp.dot(a, w_ref[...],
                           preferred_element_type=jnp.float32) + bf_ref[...]


def _tc_fin(s0, s1, g, dinv, b, wfc, bfc):
    dout = wfc.shape[1]
    return pl.pallas_call(
        _tc_fin_body,
        grid=(NBLK,),
        in_specs=[
            pl.BlockSpec((BLK, DH), lambda i: (i, 0)),
            pl.BlockSpec((BLK, DH), lambda i: (i, 0)),
            pl.BlockSpec((BLK, DH), lambda i: (i, 0)),
            pl.BlockSpec((BLK, 1), lambda i: (i, 0)),
            pl.BlockSpec((1, DH), lambda i: (0, 0)),
            pl.BlockSpec((DH, dout), lambda i: (0, 0)),
            pl.BlockSpec((1, dout), lambda i: (0, 0)),
        ],
        out_specs=pl.BlockSpec((BLK, dout), lambda i: (i, 0)),
        out_shape=jax.ShapeDtypeStruct((NPAD, dout), jnp.float32),
    )(s0, s1, g, dinv, b, wfc, bfc)


def kernel(x, edge_index, W1, b1, W2, b2, Wfc, bfc):
    src = edge_index[0].astype(jnp.int32)
    dst = edge_index[1].astype(jnp.int32)
    pad = jnp.full((EPAD - E,), N, dtype=jnp.int32)
    srcp = jnp.concatenate([src, pad]).reshape(NW, CH, K)
    dstp = jnp.concatenate([dst, pad]).reshape(NW, CH, K)
    xpad = jnp.pad(x, ((0, NPAD - N), (0, 0)))
    zeros16 = jnp.zeros((NPAD, 16), jnp.float32)
    zeros64 = jnp.zeros((NPAD, DH), jnp.float32)
    ones16 = jnp.ones((K, 16), jnp.float32)
    b1r = b1.reshape(1, DH)
    b2r = b2.reshape(1, DH)
    bfcr = bfc.reshape(1, -1)

    degp = _sc_degree(dstp, zeros16, ones16)
    dinv, g1 = _tc_pre(degp[0], degp[1], xpad, W1)
    s1 = _sc_scatter(g1, srcp, dstp, zeros64)
    g2 = _tc_mid(s1[0], s1[1], g1, dinv, b1r, W2)
    s2 = _sc_scatter(g2, srcp, dstp, zeros64)
    out = _tc_fin(s2[0], s2[1], g2, dinv, b2r, Wfc, bfcr)
    return out[:N]


# speedup vs baseline: 16.0209x; 1.1474x over previous
import functools

import jax
import jax.numpy as jnp
from jax import lax
from jax.experimental import pallas as pl
from jax.experimental.pallas import tpu as pltpu
from jax.experimental.pallas import tpu_sc as plsc

N = 10000
NPAD = 10240
E = 320000
K = 128
CH = 80
NW = 32
EPAD = NW * CH * K
RPT = NPAD // 16
BURST = 4
NBURST = CH // BURST
DH = 64
BLK = 1024
NBLK = NPAD // BLK

_mesh = plsc.VectorSubcoreMesh(core_axis_name="c", subcore_axis_name="s")
_sc_params = pltpu.CompilerParams(use_tc_tiling_on_sc=False)


@functools.partial(
    pl.kernel,
    out_type=jax.ShapeDtypeStruct((2, NPAD, 16), jnp.float32),
    mesh=_mesh,
    scratch_types=[
        pltpu.VMEM_SHARED((NPAD, 16), jnp.float32),
        pltpu.VMEM((CH, K), jnp.int32),
        pltpu.VMEM((K, 16), jnp.float32),
    ],
    compiler_params=_sc_params,
)
def _sc_degree(dst_hbm, zeros_hbm, ones_hbm, out_hbm, acc_sh, dst_v, ones_v):
    c = lax.axis_index("c")
    s = lax.axis_index("s")
    wid = c * 16 + s
    pltpu.sync_copy(dst_hbm.at[wid], dst_v)
    pltpu.sync_copy(ones_hbm, ones_v)
    pltpu.sync_copy(zeros_hbm.at[pl.ds(s * RPT, RPT)],
                    acc_sh.at[pl.ds(s * RPT, RPT)])
    plsc.subcore_barrier()

    def body(j, carry):
        pltpu.sync_copy(ones_v, acc_sh.at[dst_v.at[j]], add=True)
        return carry

    lax.fori_loop(0, CH, body, 0)
    plsc.subcore_barrier()
    pltpu.sync_copy(acc_sh.at[pl.ds(s * RPT, RPT)],
                    out_hbm.at[c, pl.ds(s * RPT, RPT)])


@functools.partial(
    pl.kernel,
    out_type=jax.ShapeDtypeStruct((2, NPAD, DH), jnp.float32),
    mesh=_mesh,
    scratch_types=[
        pltpu.VMEM_SHARED((NPAD, DH), jnp.float32),
        pltpu.VMEM((CH, K), jnp.int32),
        pltpu.VMEM((CH, K), jnp.int32),
        pltpu.VMEM((BURST, K, DH), jnp.float32),
        pltpu.VMEM((BURST, K, DH), jnp.float32),
        pltpu.SemaphoreType.DMA,
        pltpu.SemaphoreType.DMA,
    ],
    compiler_params=_sc_params,
)
def _sc_scatter(g_hbm, src_hbm, dst_hbm, zeros_hbm, out_hbm,
                acc_sh, src_v, dst_v, rows_a, rows_b, sem_a, sem_b):
    c = lax.axis_index("c")
    s = lax.axis_index("s")
    wid = c * 16 + s
    pltpu.sync_copy(src_hbm.at[wid], src_v)
    pltpu.sync_copy(dst_hbm.at[wid], dst_v)
    pltpu.sync_copy(zeros_hbm.at[pl.ds(s * RPT, RPT)],
                    acc_sh.at[pl.ds(s * RPT, RPT)])
    plsc.subcore_barrier()

    def fire(burst, bufs, sem):
        for i in range(BURST):
            pltpu.async_copy(g_hbm.at[src_v.at[burst * BURST + i]],
                             bufs.at[i], sem)

    def drain_scatter(burst, bufs, sem):
        for i in range(BURST):
            pltpu.make_async_copy(g_hbm.at[src_v.at[burst * BURST + i]],
                                  bufs.at[i], sem).wait()
            pltpu.sync_copy(bufs.at[i],
                            acc_sh.at[dst_v.at[burst * BURST + i]], add=True)

    fire(0, rows_a, sem_a)

    def body(t, carry):
        b_a = 2 * t
        b_b = 2 * t + 1
        fire(b_b, rows_b, sem_b)
        drain_scatter(b_a, rows_a, sem_a)

        @pl.when(t < NBURST // 2 - 1)
        def _():
            fire(b_a + 2, rows_a, sem_a)

        drain_scatter(b_b, rows_b, sem_b)
        return carry

    lax.fori_loop(0, NBURST // 2, body, 0)
    plsc.subcore_barrier()
    pltpu.sync_copy(acc_sh.at[pl.ds(s * RPT, RPT)],
                    out_hbm.at[c, pl.ds(s * RPT, RPT)])


def _tc_pre_body(deg0_ref, deg1_ref, x_ref, w_ref, dinv_ref, g_ref):
    i = pl.program_id(0)
    deg = deg0_ref[:, 0:1] + deg1_ref[:, 0:1] + 1.0
    rows = lax.broadcasted_iota(jnp.int32, (BLK, 1), 0) + i * BLK
    dinv = jnp.where(rows < N, lax.rsqrt(deg), 0.0)
    dinv_ref[...] = dinv
    g_ref[...] = jnp.dot(x_ref[...], w_ref[...],
                         preferred_element_type=jnp.float32) * dinv


def _tc_pre(deg0, deg1, x, w1):
    din = x.shape[1]
    return pl.pallas_call(
        _tc_pre_body,
        grid=(NBLK,),
        in_specs=[
            pl.BlockSpec((BLK, 16), lambda i: (i, 0)),
            pl.BlockSpec((BLK, 16), lambda i: (i, 0)),
            pl.BlockSpec((BLK, din), lambda i: (i, 0)),
            pl.BlockSpec((din, DH), lambda i: (0, 0)),
        ],
        out_specs=[
            pl.BlockSpec((BLK, 1), lambda i: (i, 0)),
            pl.BlockSpec((BLK, DH), lambda i: (i, 0)),
        ],
        out_shape=[
            jax.ShapeDtypeStruct((NPAD, 1), jnp.float32),
            jax.ShapeDtypeStruct((NPAD, DH), jnp.float32),
        ],
    )(deg0, deg1, x, w1)


def _tc_mid_body(s0_ref, s1_ref, g_ref, dinv_ref, b_ref, w_ref, gout_ref):
    dinv = dinv_ref[...]
    a = jnp.maximum(
        dinv * (s0_ref[...] + s1_ref[...] + g_ref[...]) + b_ref[...], 0.0)
    gout_ref[...] = jnp.dot(a, w_ref[...],
                            preferred_element_type=jnp.float32) * dinv


def _tc_mid(s0, s1, g, dinv, b, w2):
    return pl.pallas_call(
        _tc_mid_body,
        grid=(NBLK,),
        in_specs=[
            pl.BlockSpec((BLK, DH), lambda i: (i, 0)),
            pl.BlockSpec((BLK, DH), lambda i: (i, 0)),
            pl.BlockSpec((BLK, DH), lambda i: (i, 0)),
            pl.BlockSpec((BLK, 1), lambda i: (i, 0)),
            pl.BlockSpec((1, DH), lambda i: (0, 0)),
            pl.BlockSpec((DH, DH), lambda i: (0, 0)),
        ],
        out_specs=pl.BlockSpec((BLK, DH), lambda i: (i, 0)),
        out_shape=jax.ShapeDtypeStruct((NPAD, DH), jnp.float32),
    )(s0, s1, g, dinv, b, w2)


def _tc_fin_body(s0_ref, s1_ref, g_ref, dinv_ref, b_ref, w_ref, bf_ref,
                 out_ref):
    a = jnp.maximum(
        dinv_ref[...] * (s0_ref[...] + s1_ref[...] + g_ref[...]) + b_ref[...],
        0.0)
    out_ref[...] = jnp.dot(a, w_ref[...],
                           preferred_element_type=jnp.float32) + bf_ref[...]


def _tc_fin(s0, s1, g, dinv, b, wfc, bfc):
    dout = wfc.shape[1]
    return pl.pallas_call(
        _tc_fin_body,
        grid=(NBLK,),
        in_specs=[
            pl.BlockSpec((BLK, DH), lambda i: (i, 0)),
            pl.BlockSpec((BLK, DH), lambda i: (i, 0)),
            pl.BlockSpec((BLK, DH), lambda i: (i, 0)),
            pl.BlockSpec((BLK, 1), lambda i: (i, 0)),
            pl.BlockSpec((1, DH), lambda i: (0, 0)),
            pl.BlockSpec((DH, dout), lambda i: (0, 0)),
            pl.BlockSpec((1, dout), lambda i: (0, 0)),
        ],
        out_specs=pl.BlockSpec((BLK, dout), lambda i: (i, 0)),
        out_shape=jax.ShapeDtypeStruct((NPAD, dout), jnp.float32),
    )(s0, s1, g, dinv, b, wfc, bfc)


def kernel(x, edge_index, W1, b1, W2, b2, Wfc, bfc):
    src = edge_index[0].astype(jnp.int32)
    dst = edge_index[1].astype(jnp.int32)
    pad = jnp.full((EPAD - E,), N, dtype=jnp.int32)
    srcp = jnp.concatenate([src, pad]).reshape(NW, CH, K)
    dstp = jnp.concatenate([dst, pad]).reshape(NW, CH, K)
    xpad = jnp.pad(x, ((0, NPAD - N), (0, 0)))
    zeros16 = jnp.zeros((NPAD, 16), jnp.float32)
    zeros64 = jnp.zeros((NPAD, DH), jnp.float32)
    ones16 = jnp.ones((K, 16), jnp.float32)
    b1r = b1.reshape(1, DH)
    b2r = b2.reshape(1, DH)
    bfcr = bfc.reshape(1, -1)

    degp = _sc_degree(dstp, zeros16, ones16)
    dinv, g1 = _tc_pre(degp[0], degp[1], xpad, W1)
    s1 = _sc_scatter(g1, srcp, dstp, zeros64)
    g2 = _tc_mid(s1[0], s1[1], g1, dinv, b1r, W2)
    s2 = _sc_scatter(g2, srcp, dstp, zeros64)
    out = _tc_fin(s2[0], s2[1], g2, dinv, b2r, Wfc, bfcr)
    return out[:N]


# speedup vs baseline: 31.3037x; 1.9539x over previous
import functools

import jax
import jax.numpy as jnp
from jax import lax
from jax.experimental import pallas as pl
from jax.experimental.pallas import tpu as pltpu
from jax.experimental.pallas import tpu_sc as plsc

N = 10000
NPAD = 10240
E = 320000
K = 128
CH = 80
NW = 32
EPAD = NW * CH * K
NSC = 10016
RPT = NSC // 16
BURST = 1
NBURST = CH // BURST
DH = 64
BLK = 1024
NBLK = NPAD // BLK

_mesh = plsc.VectorSubcoreMesh(core_axis_name="c", subcore_axis_name="s")
_sc_params = pltpu.CompilerParams(use_tc_tiling_on_sc=False,
                                  internal_scratch_in_bytes=1 << 20)


@functools.partial(
    pl.kernel,
    out_type=jax.ShapeDtypeStruct((2, NPAD, 16), jnp.float32),
    mesh=_mesh,
    scratch_types=[
        pltpu.VMEM_SHARED((NSC, 16), jnp.float32),
        pltpu.VMEM((CH, K), jnp.int32),
        pltpu.VMEM((K, 16), jnp.float32),
    ],
    compiler_params=_sc_params,
)
def _sc_degree(dst_hbm, zeros_hbm, ones_hbm, out_hbm, acc_sh, dst_v, ones_v):
    c = lax.axis_index("c")
    s = lax.axis_index("s")
    wid = c * 16 + s
    pltpu.sync_copy(dst_hbm.at[wid], dst_v)
    pltpu.sync_copy(ones_hbm, ones_v)
    pltpu.sync_copy(zeros_hbm.at[pl.ds(s * RPT, RPT)],
                    acc_sh.at[pl.ds(s * RPT, RPT)])
    plsc.subcore_barrier()

    def body(j, carry):
        pltpu.sync_copy(ones_v, acc_sh.at[dst_v.at[j]], add=True)
        return carry

    lax.fori_loop(0, CH, body, 0)
    plsc.subcore_barrier()
    pltpu.sync_copy(acc_sh.at[pl.ds(s * RPT, RPT)],
                    out_hbm.at[c, pl.ds(s * RPT, RPT)])


@functools.partial(
    pl.kernel,
    out_type=jax.ShapeDtypeStruct((2, NPAD, DH), jnp.float32),
    mesh=_mesh,
    scratch_types=[
        pltpu.VMEM_SHARED((NSC, DH), jnp.float32),
        pltpu.VMEM_SHARED((NSC, DH), jnp.float32),
        pltpu.VMEM((CH, K), jnp.int32),
        pltpu.VMEM((CH, K), jnp.int32),
        pltpu.VMEM((BURST, K, DH), jnp.float32),
        pltpu.VMEM((BURST, K, DH), jnp.float32),
        pltpu.SemaphoreType.DMA,
        pltpu.SemaphoreType.DMA,
    ],
    compiler_params=_sc_params,
)
def _sc_scatter(g_hbm, src_hbm, dst_hbm, zeros_hbm, out_hbm,
                acc_sh, g_sh, src_v, dst_v, rows_a, rows_b, sem_a, sem_b):
    c = lax.axis_index("c")
    s = lax.axis_index("s")
    wid = c * 16 + s
    pltpu.sync_copy(src_hbm.at[wid], src_v)
    pltpu.sync_copy(dst_hbm.at[wid], dst_v)
    pltpu.sync_copy(zeros_hbm.at[pl.ds(s * RPT, RPT)],
                    acc_sh.at[pl.ds(s * RPT, RPT)])
    pltpu.sync_copy(g_hbm.at[pl.ds(s * RPT, RPT)],
                    g_sh.at[pl.ds(s * RPT, RPT)])
    plsc.subcore_barrier()

    def fire(burst, bufs, sem):
        for i in range(BURST):
            pltpu.async_copy(g_sh.at[src_v.at[burst * BURST + i]],
                             bufs.at[i], sem)

    def drain_scatter(burst, bufs, sem):
        for i in range(BURST):
            pltpu.make_async_copy(g_sh.at[src_v.at[burst * BURST + i]],
                                  bufs.at[i], sem).wait()
            pltpu.sync_copy(bufs.at[i],
                            acc_sh.at[dst_v.at[burst * BURST + i]], add=True)

    fire(0, rows_a, sem_a)

    def body(t, carry):
        b_a = 2 * t
        b_b = 2 * t + 1
        fire(b_b, rows_b, sem_b)
        drain_scatter(b_a, rows_a, sem_a)

        @pl.when(t < NBURST // 2 - 1)
        def _():
            fire(b_a + 2, rows_a, sem_a)

        drain_scatter(b_b, rows_b, sem_b)
        return carry

    lax.fori_loop(0, NBURST // 2, body, 0)
    plsc.subcore_barrier()
    pltpu.sync_copy(acc_sh.at[pl.ds(s * RPT, RPT)],
                    out_hbm.at[c, pl.ds(s * RPT, RPT)])


def _tc_pre_body(deg0_ref, deg1_ref, x_ref, w_ref, dinv_ref, g_ref):
    i = pl.program_id(0)
    deg = deg0_ref[:, 0:1] + deg1_ref[:, 0:1] + 1.0
    rows = lax.broadcasted_iota(jnp.int32, (BLK, 1), 0) + i * BLK
    dinv = jnp.where(rows < N, lax.rsqrt(deg), 0.0)
    dinv_ref[...] = dinv
    g_ref[...] = jnp.dot(x_ref[...], w_ref[...],
                         preferred_element_type=jnp.float32) * dinv


def _tc_pre(deg0, deg1, x, w1):
    din = x.shape[1]
    return pl.pallas_call(
        _tc_pre_body,
        grid=(NBLK,),
        in_specs=[
            pl.BlockSpec((BLK, 16), lambda i: (i, 0)),
            pl.BlockSpec((BLK, 16), lambda i: (i, 0)),
            pl.BlockSpec((BLK, din), lambda i: (i, 0)),
            pl.BlockSpec((din, DH), lambda i: (0, 0)),
        ],
        out_specs=[
            pl.BlockSpec((BLK, 1), lambda i: (i, 0)),
            pl.BlockSpec((BLK, DH), lambda i: (i, 0)),
        ],
        out_shape=[
            jax.ShapeDtypeStruct((NPAD, 1), jnp.float32),
            jax.ShapeDtypeStruct((NPAD, DH), jnp.float32),
        ],
    )(deg0, deg1, x, w1)


def _tc_mid_body(s0_ref, s1_ref, g_ref, dinv_ref, b_ref, w_ref, gout_ref):
    dinv = dinv_ref[...]
    a = jnp.maximum(
        dinv * (s0_ref[...] + s1_ref[...] + g_ref[...]) + b_ref[...], 0.0)
    gout_ref[...] = jnp.dot(a, w_ref[...],
                            preferred_element_type=jnp.float32) * dinv


def _tc_mid(s0, s1, g, dinv, b, w2):
    return pl.pallas_call(
        _tc_mid_body,
        grid=(NBLK,),
        in_specs=[
            pl.BlockSpec((BLK, DH), lambda i: (i, 0)),
            pl.BlockSpec((BLK, DH), lambda i: (i, 0)),
            pl.BlockSpec((BLK, DH), lambda i: (i, 0)),
            pl.BlockSpec((BLK, 1), lambda i: (i, 0)),
            pl.BlockSpec((1, DH), lambda i: (0, 0)),
            pl.BlockSpec((DH, DH), lambda i: (0, 0)),
        ],
        out_specs=pl.BlockSpec((BLK, DH), lambda i: (i, 0)),
        out_shape=jax.ShapeDtypeStruct((NPAD, DH), jnp.float32),
    )(s0, s1, g, dinv, b, w2)


def _tc_fin_body(s0_ref, s1_ref, g_ref, dinv_ref, b_ref, w_ref, bf_ref,
                 out_ref):
    a = jnp.maximum(
        dinv_ref[...] * (s0_ref[...] + s1_ref[...] + g_ref[...]) + b_ref[...],
        0.0)
    out_ref[...] = jnp.dot(a, w_ref[...],
                           preferred_element_type=jnp.float32) + bf_ref[...]


def _tc_fin(s0, s1, g, dinv, b, wfc, bfc):
    dout = wfc.shape[1]
    return pl.pallas_call(
        _tc_fin_body,
        grid=(NBLK,),
        in_specs=[
            pl.BlockSpec((BLK, DH), lambda i: (i, 0)),
            pl.BlockSpec((BLK, DH), lambda i: (i, 0)),
            pl.BlockSpec((BLK, DH), lambda i: (i, 0)),
            pl.BlockSpec((BLK, 1), lambda i: (i, 0)),
            pl.BlockSpec((1, DH), lambda i: (0, 0)),
            pl.BlockSpec((DH, dout), lambda i: (0, 0)),
            pl.BlockSpec((1, dout), lambda i: (0, 0)),
        ],
        out_specs=pl.BlockSpec((BLK, dout), lambda i: (i, 0)),
        out_shape=jax.ShapeDtypeStruct((NPAD, dout), jnp.float32),
    )(s0, s1, g, dinv, b, wfc, bfc)


def kernel(x, edge_index, W1, b1, W2, b2, Wfc, bfc):
    src = edge_index[0].astype(jnp.int32)
    dst = edge_index[1].astype(jnp.int32)
    pad = jnp.full((EPAD - E,), N, dtype=jnp.int32)
    srcp = jnp.concatenate([src, pad]).reshape(NW, CH, K)
    dstp = jnp.concatenate([dst, pad]).reshape(NW, CH, K)
    xpad = jnp.pad(x, ((0, NPAD - N), (0, 0)))
    zeros16 = jnp.zeros((NPAD, 16), jnp.float32)
    zeros64 = jnp.zeros((NPAD, DH), jnp.float32)
    ones16 = jnp.ones((K, 16), jnp.float32)
    b1r = b1.reshape(1, DH)
    b2r = b2.reshape(1, DH)
    bfcr = bfc.reshape(1, -1)

    degp = _sc_degree(dstp, zeros16, ones16)
    dinv, g1 = _tc_pre(degp[0], degp[1], xpad, W1)
    s1 = _sc_scatter(g1, srcp, dstp, zeros64)
    g2 = _tc_mid(s1[0], s1[1], g1, dinv, b1r, W2)
    s2 = _sc_scatter(g2, srcp, dstp, zeros64)
    out = _tc_fin(s2[0], s2[1], g2, dinv, b2r, Wfc, bfcr)
    return out[:N]
